# baseline (speedup 1.0000x reference)
import numpy as np

import jax
import jax.numpy as jnp
from jax import lax
from jax.experimental import pallas as pl
from jax.experimental.pallas import tpu as pltpu

_NC = 8192
_NOBJ = 32
_DIMC = _NC + _NOBJ
_D = 64
_S = 16
_B = 32
_P = 4
_CHUNK = 2048
_NCHUNK = _NC // _CHUNK

_ROTS = [[13, 15, 26, 6], [17, 29, 16, 24]]


def _rol(x, d):
    return (lax.shift_left(x, jnp.int32(d))
            | lax.shift_right_logical(x, jnp.int32(32 - d)))


def _gumbel_block(key_consts, b_abs):
    k0c, k1c = key_consts
    k0 = jnp.int32(k0c)
    k1 = jnp.int32(k1c)
    ks2 = k0 ^ k1 ^ jnp.int32(0x1BD11BDA)
    ks = [k0, k1, ks2]
    cnt = (lax.shift_left(lax.broadcasted_iota(jnp.int32, (_S, _NC), 0),
                          jnp.int32(18))
           + lax.broadcasted_iota(jnp.int32, (_S, _NC), 1)
           + b_abs * _NC)
    x0 = jnp.zeros((_S, _NC), jnp.int32) + ks[0]
    x1 = cnt + ks[1]
    for i in range(5):
        for r in _ROTS[i % 2]:
            x0 = x0 + x1
            x1 = _rol(x1, r)
            x1 = x1 ^ x0
        x0 = x0 + ks[(i + 1) % 3]
        x1 = x1 + ks[(i + 2) % 3] + jnp.int32(i + 1)
    bits = x0 ^ x1
    f = lax.bitcast_convert_type(
        lax.shift_right_logical(bits, jnp.int32(9)) | jnp.int32(0x3F800000),
        jnp.float32)
    u = f - jnp.float32(1.0)
    tiny = jnp.float32(np.finfo(np.float32).tiny)
    u = jnp.maximum(tiny, u * (jnp.float32(1.0) - tiny) + tiny)
    return -jnp.log(-jnp.log(u))


def _sample_and_mlp(g, logits, ctT, gather_col, oparg, w1t, b1c, w2t, b2c):
    v = g if logits is None else g + logits
    m = jnp.max(v, axis=1, keepdims=True)
    iota = lax.broadcasted_iota(jnp.int32, (_S, _NC), 1)
    idx = jnp.min(jnp.where(v == m, iota, _NC), axis=1, keepdims=True)
    idxf = idx.astype(jnp.float32)
    eye = (lax.broadcasted_iota(jnp.int32, (_S, _S), 0)
           == lax.broadcasted_iota(jnp.int32, (_S, _S), 1)).astype(jnp.float32)
    sel_row = jnp.dot(jnp.ones((1, _S), jnp.float32), eye * idxf,
                      preferred_element_type=jnp.float32)
    onehotT = (lax.broadcasted_iota(jnp.int32, (_NC, _S), 0)
               == sel_row.astype(jnp.int32)).astype(jnp.float32)
    tout_selT = jnp.dot(ctT, onehotT,
                        preferred_element_type=jnp.float32)
    gb = jnp.broadcast_to(gather_col, (_D, _S))
    xT = jnp.concatenate([tout_selT, gb, oparg], axis=0)
    hT = jax.nn.relu(jnp.dot(w1t, xT, preferred_element_type=jnp.float32)
                     + b1c)
    axonT = jnp.dot(w2t, hT, preferred_element_type=jnp.float32) + b2c
    return tout_selT, axonT


def _mT_of(aT, sT):
    return lax.dot_general(sT, aT, (((1,), (1,)), ((), ())),
                           preferred_element_type=jnp.float32) * (1.0 / (_D * _S))


def _stats(mT, ctT, objT, scal_ref, k):
    acc = jnp.zeros((_D, _CHUNK), jnp.float32)
    for c in range(_NCHUNK):
        attT = jax.nn.relu(jnp.dot(mT, ctT[:, c * _CHUNK:(c + 1) * _CHUNK],
                                   preferred_element_type=jnp.float32))
        scal_ref[k:k + 1, c * _CHUNK:(c + 1) * _CHUNK] = jnp.dot(
            jnp.ones((1, _D), jnp.float32), attT,
            preferred_element_type=jnp.float32)
        acc = acc + attT
    attT_obj = jax.nn.relu(jnp.dot(mT, objT,
                                   preferred_element_type=jnp.float32))
    csum = (jnp.sum(acc, axis=1, keepdims=True)
            + jnp.sum(attT_obj, axis=1, keepdims=True))
    return csum * (1.0 / _DIMC)


def _step0_body(key_consts, ctT_ref, oparg_ref, w1t_ref, b1_ref, w2t_ref,
                b2_ref, init_ref, axonT_out, attselT_out):
    ctT = ctT_ref[...]
    initcol = init_ref[...]
    j = pl.program_id(0)
    for k in range(_P):
        g = _gumbel_block(key_consts, j * _P + k)
        _, axonT = _sample_and_mlp(
            g, None, ctT, initcol, oparg_ref[k], w1t_ref[...], b1_ref[...],
            w2t_ref[...], b2_ref[...])
        axonT_out[k] = axonT
        attselT_out[k] = jnp.broadcast_to(initcol, (_D, _S))


def _step_body(key_consts, ctT_ref, objT_ref, axonT_ref, attselT_ref,
               oparg_ref, w1t_ref, b1_ref, w2t_ref, b2_ref,
               axonT_out, attselT_out, scal_ref):
    ctT = ctT_ref[...]
    j = pl.program_id(0)
    for k in range(_P):
        mT = _mT_of(axonT_ref[k], attselT_ref[k])
        gather_col = _stats(mT, ctT, objT_ref[k], scal_ref, k)
        scal = scal_ref[k:k + 1, :]
        ssum = jnp.sum(scal)
        logits = jnp.log(scal / ssum + 1e-12)
        g = _gumbel_block(key_consts, j * _P + k)
        tout_selT, axonT = _sample_and_mlp(
            g, logits, ctT, gather_col, oparg_ref[k], w1t_ref[...],
            b1_ref[...], w2t_ref[...], b2_ref[...])
        axonT_out[k] = axonT
        attselT_out[k] = jax.nn.relu(jnp.dot(mT, tout_selT,
                                             preferred_element_type=jnp.float32))


def _final_body(ctT_ref, objT_ref, axonT_ref, attselT_ref, out_ref, len_ref):
    ctT = ctT_ref[...]
    inv = jnp.ones((1, _D), jnp.float32) * (1.0 / _D)
    for k in range(_P):
        mT = _mT_of(axonT_ref[k], attselT_ref[k])
        for c in range(_NCHUNK):
            attT = jax.nn.relu(jnp.dot(mT, ctT[:, c * _CHUNK:(c + 1) * _CHUNK],
                                       preferred_element_type=jnp.float32))
            len_ref[k:k + 1, c * _CHUNK:(c + 1) * _CHUNK] = jnp.dot(
                inv, attT * attT, preferred_element_type=jnp.float32)
        attT_obj = jax.nn.relu(jnp.dot(mT, objT_ref[k],
                                       preferred_element_type=jnp.float32))
        len_ref[k:k + 1, _NC:] = jnp.dot(inv, attT_obj * attT_obj,
                                         preferred_element_type=jnp.float32)
    x = len_ref[...]
    m = jnp.max(x, axis=1, keepdims=True)
    sh = x - m
    out_ref[0] = sh - jnp.log(jnp.sum(jnp.exp(sh), axis=1, keepdims=True))


def _np_threefry2x32(k0, k1, x0, x1):
    M = 0xFFFFFFFF
    rotl = lambda x, d: ((x << d) | (x >> (32 - d))) & M
    ks = [k0, k1, k0 ^ k1 ^ 0x1BD11BDA]
    x0 = (x0 + ks[0]) & M
    x1 = (x1 + ks[1]) & M
    for i in range(5):
        for r in _ROTS[i % 2]:
            x0 = (x0 + x1) & M
            x1 = rotl(x1, r)
            x1 ^= x0
        x0 = (x0 + ks[(i + 1) % 3]) & M
        x1 = (x1 + ks[(i + 2) % 3] + i + 1) & M
    return x0, x1


def _key_consts(i):
    k0, k1 = _np_threefry2x32(0, 42, 0, i)
    to_i32 = lambda v: v - (1 << 32) if v >= (1 << 31) else v
    return (to_i32(k0), to_i32(k1))


def kernel(gt_classes, gt_attributes, program, answer, class_emb_in,
           class_emb_out, attr_emb_in, attr_emb_out, concept_emb_in,
           concept_emb_out, op_emb, object_init, attention_init, W1, b1,
           W2, b2):
    del answer, class_emb_out, attr_emb_out, object_init
    f32 = jnp.float32
    B = _B
    NG = B // _P

    non_bg = (gt_attributes != -1).astype(f32)
    obj_in = jnp.take(class_emb_in, gt_classes + 1, axis=0) + \
        (jnp.take(attr_emb_in, gt_attributes + 1, axis=0)
         * non_bg[..., None]).sum(2)
    objT = jnp.transpose(obj_in, (0, 2, 1))
    ctT = concept_emb_in.T
    operations = jnp.take(op_emb, program[:, :, 0], axis=0)
    arguments = jnp.take(concept_emb_out, program[:, :, 1], axis=0)
    opargs = jnp.concatenate([operations, arguments], axis=2)
    opargs = jnp.broadcast_to(opargs[..., None], (B, 4, 96, _S))
    w1t, w2t = W1.T, W2.T
    b1c, b2c = b1[:, None], b2[:, None]
    initcol = attention_init[:, None]

    keys = [_key_consts(i) for i in range(4)]

    const_spec = pl.BlockSpec((_D, _NC), lambda i: (0, 0))
    slab3 = lambda shp: pl.BlockSpec(shp, lambda i: (i, 0, 0))
    full = lambda shp: pl.BlockSpec(shp, lambda i: (0,) * len(shp))
    state_shape = jax.ShapeDtypeStruct((B, _D, _S), f32)
    state_spec = slab3((_P, _D, _S))

    import functools
    step0 = pl.pallas_call(
        functools.partial(_step0_body, keys[0]),
        grid=(NG,),
        in_specs=[const_spec, slab3((_P, 96, _S)),
                  full((256, 224)), full((256, 1)), full((64, 256)),
                  full((64, 1)), full((_D, 1))],
        out_specs=[state_spec, state_spec],
        out_shape=[state_shape, state_shape],
    )
    axonT, attselT = step0(ctT, opargs[:, 0], w1t, b1c, w2t, b2c, initcol)

    for i in range(1, 4):
        step = pl.pallas_call(
            functools.partial(_step_body, keys[i]),
            grid=(NG,),
            in_specs=[const_spec, slab3((_P, _D, _NOBJ)),
                      state_spec, state_spec, slab3((_P, 96, _S)),
                      full((256, 224)), full((256, 1)), full((64, 256)),
                      full((64, 1))],
            out_specs=[state_spec, state_spec],
            out_shape=[state_shape, state_shape],
            scratch_shapes=[pltpu.VMEM((_P, _NC), f32)],
        )
        axonT, attselT = step(ctT, objT, axonT, attselT,
                              opargs[:, i], w1t, b1c, w2t, b2c)

    final = pl.pallas_call(
        _final_body,
        grid=(NG,),
        in_specs=[const_spec, slab3((_P, _D, _NOBJ)), state_spec, state_spec],
        out_specs=pl.BlockSpec((1, _P, _DIMC), lambda i: (i, 0, 0)),
        out_shape=jax.ShapeDtypeStruct((NG, _P, _DIMC), f32),
        scratch_shapes=[pltpu.VMEM((_P, _DIMC), f32)],
    )
    return final(ctT, objT, axonT, attselT).reshape(B, _DIMC)


# speedup vs baseline: 1.6265x; 1.6265x over previous
import jax
import jax.numpy as jnp
from jax import lax
from jax.experimental import pallas as pl
from jax.experimental.pallas import tpu as pltpu

_NC = 8192
_NOBJ = 32
_DIMC = _NC + _NOBJ
_D = 64
_S = 16
_B = 32
_P = 4
_CHUNK = 2048
_NCHUNK = _NC // _CHUNK
_PD = _P * _D
_PS = _P * _S


def _eye(n):
    return (lax.broadcasted_iota(jnp.int32, (n, n), 0)
            == lax.broadcasted_iota(jnp.int32, (n, n), 1)).astype(jnp.float32)


def _mT_of(a, s):
    return lax.dot_general(s, a, (((0,), (0,)), ((), ())),
                           preferred_element_type=jnp.float32) * (1.0 / (_D * _S))


def _sample(g, logits):
    v = g if logits is None else g + logits
    m = jnp.max(v, axis=1, keepdims=True)
    iota = lax.broadcasted_iota(jnp.int32, (_S, _NC), 1)
    idx = jnp.min(jnp.where(v == m, iota, _NC), axis=1, keepdims=True)
    return (iota == idx).astype(jnp.float32)


def _mlp(x_all, w1, b1r, w2, b2r):
    h = jax.nn.relu(jnp.dot(x_all, w1, preferred_element_type=jnp.float32)
                    + b1r)
    return jnp.dot(h, w2, preferred_element_type=jnp.float32) + b2r


def _step0_body(ctT_ref, ct_ref, g_ref, oparg_ref, w1_ref, b1_ref, w2_ref,
                b2_ref, init_ref, axon_out, attsel_out):
    del ctT_ref
    initrow = init_ref[...]
    onehots = [_sample(g_ref[:, k * _NC:(k + 1) * _NC], None)
               for k in range(_P)]
    tout_all = jnp.dot(jnp.concatenate(onehots, axis=0), ct_ref[...],
                       preferred_element_type=jnp.float32)
    gb = jnp.broadcast_to(initrow, (_PS, _D))
    x_all = jnp.concatenate([tout_all, gb, oparg_ref[0]], axis=1)
    axon_all = _mlp(x_all, w1_ref[...], b1_ref[...], w2_ref[...], b2_ref[...])
    for k in range(_P):
        axon_out[k] = axon_all[k * _S:(k + 1) * _S]
        attsel_out[k] = jnp.broadcast_to(initrow, (_S, _D))


def _step_body(ctT_ref, ct_ref, g_ref, objT_ref, axon_ref, attsel_ref,
               oparg_ref, w1_ref, b1_ref, w2_ref, b2_ref,
               axon_out, attsel_out, scal_ref):
    ctT = ctT_ref[...]
    mTs = [_mT_of(axon_ref[k], attsel_ref[k]) for k in range(_P)]
    mT_all = jnp.concatenate(mTs, axis=0)
    ones_row = jnp.ones((1, _D), jnp.float32)
    acc = jnp.zeros((_PD, _CHUNK), jnp.float32)
    for c in range(_NCHUNK):
        attT_all = jax.nn.relu(jnp.dot(mT_all,
                                       ctT[:, c * _CHUNK:(c + 1) * _CHUNK],
                                       preferred_element_type=jnp.float32))
        for k in range(_P):
            scal_ref[k:k + 1, c * _CHUNK:(c + 1) * _CHUNK] = jnp.dot(
                ones_row, attT_all[k * _D:(k + 1) * _D],
                preferred_element_type=jnp.float32)
        acc = acc + attT_all
    rowtot = jnp.sum(acc, axis=1, keepdims=True)
    eye = _eye(_D)
    onehots, grows = [], []
    for k in range(_P):
        attT_obj = jax.nn.relu(jnp.dot(mTs[k], objT_ref[k],
                                       preferred_element_type=jnp.float32))
        gcol = (rowtot[k * _D:(k + 1) * _D]
                + jnp.sum(attT_obj, axis=1, keepdims=True)) * (1.0 / _DIMC)
        grows.append(lax.dot_general(gcol, eye, (((0,), (0,)), ((), ())),
                                     preferred_element_type=jnp.float32))
        scal = scal_ref[k:k + 1, :]
        logits = jnp.log(scal / jnp.sum(scal) + 1e-12)
        onehots.append(_sample(g_ref[:, k * _NC:(k + 1) * _NC], logits))
    tout_all = jnp.dot(jnp.concatenate(onehots, axis=0), ct_ref[...],
                       preferred_element_type=jnp.float32)
    xs = []
    for k in range(_P):
        tout_k = tout_all[k * _S:(k + 1) * _S]
        attsel_out[k] = jax.nn.relu(
            lax.dot_general(tout_k, mTs[k], (((1,), (1,)), ((), ())),
                            preferred_element_type=jnp.float32))
        xs.append(jnp.concatenate(
            [tout_k, jnp.broadcast_to(grows[k], (_S, _D)),
             oparg_ref[0][k * _S:(k + 1) * _S]], axis=1))
    x_all = jnp.concatenate(xs, axis=0)
    axon_all = _mlp(x_all, w1_ref[...], b1_ref[...], w2_ref[...], b2_ref[...])
    for k in range(_P):
        axon_out[k] = axon_all[k * _S:(k + 1) * _S]


def _final_body(ctT_ref, objT_ref, axon_ref, attsel_ref, out_ref, len_ref):
    ctT = ctT_ref[...]
    mTs = [_mT_of(axon_ref[k], attsel_ref[k]) for k in range(_P)]
    mT_all = jnp.concatenate(mTs, axis=0)
    inv = jnp.ones((1, _D), jnp.float32) * (1.0 / _D)
    for c in range(_NCHUNK):
        attT_all = jax.nn.relu(jnp.dot(mT_all,
                                       ctT[:, c * _CHUNK:(c + 1) * _CHUNK],
                                       preferred_element_type=jnp.float32))
        sq = attT_all * attT_all
        for k in range(_P):
            len_ref[k:k + 1, c * _CHUNK:(c + 1) * _CHUNK] = jnp.dot(
                inv, sq[k * _D:(k + 1) * _D],
                preferred_element_type=jnp.float32)
    for k in range(_P):
        attT_obj = jax.nn.relu(jnp.dot(mTs[k], objT_ref[k],
                                       preferred_element_type=jnp.float32))
        len_ref[k:k + 1, _NC:] = jnp.dot(inv, attT_obj * attT_obj,
                                         preferred_element_type=jnp.float32)
    x = len_ref[...]
    m = jnp.max(x, axis=1, keepdims=True)
    sh = x - m
    out_ref[0] = sh - jnp.log(jnp.sum(jnp.exp(sh), axis=1, keepdims=True))


def kernel(gt_classes, gt_attributes, program, answer, class_emb_in,
           class_emb_out, attr_emb_in, attr_emb_out, concept_emb_in,
           concept_emb_out, op_emb, object_init, attention_init, W1, b1,
           W2, b2):
    del answer, class_emb_out, attr_emb_out, object_init
    f32 = jnp.float32
    B = _B
    NG = B // _P

    non_bg = (gt_attributes != -1).astype(f32)
    obj_in = jnp.take(class_emb_in, gt_classes + 1, axis=0) + \
        (jnp.take(attr_emb_in, gt_attributes + 1, axis=0)
         * non_bg[..., None]).sum(2)
    objT = jnp.transpose(obj_in, (0, 2, 1))
    ctT = concept_emb_in.T
    operations = jnp.take(op_emb, program[:, :, 0], axis=0)
    arguments = jnp.take(concept_emb_out, program[:, :, 1], axis=0)
    opargs = jnp.concatenate([operations, arguments], axis=2)
    opargs = jnp.broadcast_to(
        opargs.transpose(1, 0, 2)[:, :, None, :], (4, B, _S, 96)
    ).reshape(4, NG, _PS, 96)
    b1r, b2r = b1[None], b2[None]
    initrow = attention_init[None]

    skey = jax.random.key(42)
    gs = [jax.random.gumbel(jax.random.fold_in(skey, i), (_S, B * _NC), f32)
          for i in range(4)]

    const_spec = pl.BlockSpec((_D, _NC), lambda i: (0, 0))
    ct_spec = pl.BlockSpec((_NC, _D), lambda i: (0, 0))
    g_spec = pl.BlockSpec((_S, _P * _NC), lambda i: (0, i))
    slab3 = lambda shp: pl.BlockSpec(shp, lambda i: (i, 0, 0))
    full = lambda shp: pl.BlockSpec(shp, lambda i: (0,) * len(shp))
    state_shape = jax.ShapeDtypeStruct((B, _S, _D), f32)
    state_spec = slab3((_P, _S, _D))
    oparg_spec = slab3((1, _PS, 96))

    step0 = pl.pallas_call(
        _step0_body,
        grid=(NG,),
        in_specs=[const_spec, ct_spec, g_spec, oparg_spec,
                  full((224, 256)), full((1, 256)), full((256, 64)),
                  full((1, 64)), full((1, _D))],
        out_specs=[state_spec, state_spec],
        out_shape=[state_shape, state_shape],
    )
    axon, attsel = step0(ctT, concept_emb_in, gs[0], opargs[0], W1, b1r,
                         W2, b2r, initrow)

    step = pl.pallas_call(
        _step_body,
        grid=(NG,),
        in_specs=[const_spec, ct_spec, g_spec, slab3((_P, _D, _NOBJ)),
                  state_spec, state_spec, oparg_spec,
                  full((224, 256)), full((1, 256)), full((256, 64)),
                  full((1, 64))],
        out_specs=[state_spec, state_spec],
        out_shape=[state_shape, state_shape],
        scratch_shapes=[pltpu.VMEM((_P, _NC), f32)],
    )
    for i in range(1, 4):
        axon, attsel = step(ctT, concept_emb_in, gs[i], objT, axon, attsel,
                            opargs[i], W1, b1r, W2, b2r)

    final = pl.pallas_call(
        _final_body,
        grid=(NG,),
        in_specs=[const_spec, slab3((_P, _D, _NOBJ)), state_spec, state_spec],
        out_specs=pl.BlockSpec((1, _P, _DIMC), lambda i: (i, 0, 0)),
        out_shape=jax.ShapeDtypeStruct((NG, _P, _DIMC), f32),
        scratch_shapes=[pltpu.VMEM((_P, _DIMC), f32)],
    )
    return final(ctT, objT, axon, attsel).reshape(B, _DIMC)
